# baseline (speedup 1.0000x reference)
import functools

import jax
import jax.numpy as jnp
from jax import lax
from jax.experimental import pallas as pl
from jax.experimental.pallas import tpu as pltpu
from jax.experimental.pallas import tpu_sc as plsc

B = 64
V = 1000000
NC = 2
NS = 16
L = 16
NW = NC * NS
ROWS_PER_W = B // NW
CH = 20000
NCHUNK = V // CH
NPAIR = NCHUNK // 2
VREGS = CH // L

_mesh = plsc.VectorSubcoreMesh(core_axis_name="c", subcore_axis_name="s")


@functools.partial(
    pl.kernel,
    out_type=jax.ShapeDtypeStruct((B * L,), jnp.int32),
    mesh=_mesh,
    scratch_types=[
        pltpu.VMEM((CH,), jnp.float32),
        pltpu.VMEM((CH,), jnp.float32),
        pltpu.VMEM((L,), jnp.int32),
        pltpu.SemaphoreType.DMA,
        pltpu.SemaphoreType.DMA,
    ],
)
def _argmax_kernel(logits, out, buf0, buf1, outv, sem0, sem1):
    wid = lax.axis_index("c") * NS + lax.axis_index("s")
    lane = lax.iota(jnp.int32, L)

    def scan_chunk(buf, base_vreg, bv, bj):
        def body(j, carry):
            cv, cj = carry
            v = buf[pl.ds(j * L, L)]
            gt = v > cv
            jv = lax.broadcast(base_vreg + j, (L,))
            return jnp.where(gt, v, cv), jnp.where(gt, jv, cj)

        return lax.fori_loop(0, VREGS, body, (bv, bj), unroll=8)

    for r in range(ROWS_PER_W):
        row = wid * ROWS_PER_W + r

        def src(c):
            off = pl.multiple_of(row * V + c * CH, 8)
            return logits.at[pl.ds(off, CH)]

        pltpu.make_async_copy(src(0), buf0, sem0).start()

        def pair_body(cc, carry):
            bv, bj = carry
            c0 = 2 * cc
            pltpu.make_async_copy(src(c0 + 1), buf1, sem1).start()
            pltpu.make_async_copy(src(c0), buf0, sem0).wait()
            bv, bj = scan_chunk(buf0, c0 * VREGS, bv, bj)

            @pl.when(cc < NPAIR - 1)
            def _():
                pltpu.make_async_copy(src(c0 + 2), buf0, sem0).start()

            pltpu.make_async_copy(src(c0 + 1), buf1, sem1).wait()
            bv, bj = scan_chunk(buf1, (c0 + 1) * VREGS, bv, bj)
            return bv, bj

        init = (jnp.full((L,), -jnp.inf, jnp.float32),
                jnp.zeros((L,), jnp.int32))
        bv, bj = lax.fori_loop(0, NPAIR, pair_body, init)

        def perm_gather(x, p):
            return lax.gather(
                x, p[:, None],
                lax.GatherDimensionNumbers(
                    offset_dims=(), collapsed_slice_dims=(0,),
                    start_index_map=(0,)),
                (1,), mode=lax.GatherScatterMode.PROMISE_IN_BOUNDS)

        m = bv
        for s in (1, 2, 4, 8):
            m = jnp.maximum(m, perm_gather(m, lane ^ s))
        idx = bj * L + lane
        cand = jnp.where(bv == m, idx, jnp.int32(2147483647))
        for s in (1, 2, 4, 8):
            cand = jnp.minimum(cand, perm_gather(cand, lane ^ s))
        outv[...] = cand
        pltpu.sync_copy(outv, out.at[pl.ds(pl.multiple_of(row * L, 8), L)])


def kernel(m_logits):
    out = _argmax_kernel(m_logits.reshape(B * V))
    return out.reshape(B, L)[:, :1]
